# baseline (speedup 1.0000x reference)
import jax
import jax.numpy as jnp
from jax import lax
from jax.experimental import pallas as pl
from jax.experimental.pallas import tpu as pltpu

_IOU_THRESH = 0.1
_ALPHA = 1.0
_BETA = 1.0
_G = 8
_QR = 16
_QL = 128
_QP = _QR * _QL


def _loss_kernel(logits_ref, pgrid_ref, pbox_ref, tbox_ref, l0_ref,
                 tflat_ref, lbl_ref, out_ref, *accs):
    G, Q, C = logits_ref.shape
    T = tbox_ref.shape[1]
    ccls_s = accs[:G]
    creg_s = accs[G:]
    tflat = tflat_ref.at[0]
    lbls = lbl_ref.at[0]

    idx_grid = (lax.broadcasted_iota(jnp.int32, (_QR, _QL), 0) * _QL
                + lax.broadcasted_iota(jnp.int32, (_QR, _QL), 1))
    lane_c = lax.broadcasted_iota(jnp.int32, (1, C), 1)

    ce0 = []
    chunk = 360 if Q % 360 == 0 else Q
    for g in range(G):
        tot = jnp.zeros((1, 1), dtype=jnp.float32)
        for q0 in range(0, Q, chunk):
            x = logits_ref[g, q0:q0 + chunk, :]
            m = jnp.max(x, axis=1, keepdims=True)
            s = jnp.sum(jnp.exp(x - m), axis=1, keepdims=True)
            lse = m + jnp.log(s)
            tot = tot + jnp.sum(lse - x[:, 0:1], keepdims=True)
        ce0.append(tot)

    for g in range(G):
        ccls_s[g][...] = jnp.zeros_like(ccls_s[g])
        creg_s[g][...] = jnp.zeros_like(creg_s[g])

    def _iou_at(t):
        tb = 4 * t
        iou_l = []
        for g in range(G):
            tx1 = tflat[g, tb]
            ty1 = tflat[g, tb + 1]
            tx2 = tflat[g, tb + 2]
            ty2 = tflat[g, tb + 3]
            area_t = (tx2 - tx1) * (ty2 - ty1)

            px1 = pgrid_ref[g, 0]
            py1 = pgrid_ref[g, 1]
            px2 = pgrid_ref[g, 2]
            py2 = pgrid_ref[g, 3]
            iw = jnp.maximum(jnp.minimum(px2, tx2)
                             - jnp.maximum(px1, tx1), 0.0)
            ih = jnp.maximum(jnp.minimum(py2, ty2)
                             - jnp.maximum(py1, ty1), 0.0)
            inter = iw * ih
            union = (px2 - px1) * (py2 - py1) + area_t - inter
            iou_l.append(inter / (union + 1e-6))
        return jnp.stack(iou_l, axis=0)

    def _tree16(x3, op):
        a = op(x3[:, 0:8], x3[:, 8:16])
        b = op(a[:, 0:4], a[:, 4:8])
        c = op(b[:, 0:2], b[:, 2:4])
        return op(c[:, 0], c[:, 1])

    def body(t, carry):
        avail3, iou3 = carry
        masked3 = jnp.where(avail3 > 0.5, iou3, 0.0)
        iou_next = _iou_at(jnp.minimum(t + 1, T - 1))
        bmax = jnp.max(_tree16(masked3, jnp.maximum), axis=1,
                       keepdims=True)
        matched = bmax > _IOU_THRESH
        cand = masked3 == bmax[:, :, None]
        idxm = jnp.where(cand, idx_grid[None], _QP)
        qmin = jnp.min(_tree16(idxm, jnp.minimum), axis=1,
                       keepdims=True)
        qeff = jnp.where(matched, qmin, _QP)
        avail_new = jnp.where(idx_grid[None] == qeff[:, :, None],
                              0.0, avail3)

        for g in range(G):
            q = jnp.minimum(qmin[g, 0], Q - 1)
            mg = matched[g:g + 1, 0:1]
            lbl = lbls[g, t]
            row = logits_ref[g, pl.ds(q, 1), :]
            contrib = jnp.where(lane_c == lbl, row, 0.0)
            ccls_s[g][...] = ccls_s[g][...] - jnp.where(mg, contrib, 0.0)

            bp = pbox_ref[g, pl.ds(q, 1), :]
            tbx = tbox_ref[g, pl.ds(t, 1), :]
            d = jnp.abs(bp - tbx)
            sl1 = jnp.where(d < 1.0, 0.5 * d * d, d - 0.5)
            creg_s[g][...] = creg_s[g][...] + jnp.where(mg, sl1, 0.0)
        return (avail_new, iou_next)

    init = (jnp.ones((G, _QR, _QL), dtype=jnp.float32), _iou_at(0))
    avail3_fin, _ = lax.fori_loop(0, T, body, init)
    avail_fin = [avail3_fin[g] for g in range(G)]

    cls_tot = jnp.zeros((1, 1), dtype=jnp.float32)
    reg_tot = jnp.zeros((1, 1), dtype=jnp.float32)
    for g in range(G):
        l0_assigned = jnp.sum(
            jnp.where(avail_fin[g] < 0.5, l0_ref[g], 0.0), keepdims=True)
        cls_tot = (cls_tot + ce0[g] + l0_assigned
                   + jnp.sum(ccls_s[g][...], axis=1, keepdims=True))
        reg_tot = reg_tot + jnp.sum(creg_s[g][...], axis=1,
                                    keepdims=True) * 0.25
    out_ref[0] = jnp.concatenate([cls_tot, reg_tot], axis=1)


@jax.jit
def kernel(pred_logits, pred_boxes, target_boxes, target_labels):
    B, Q, C = pred_logits.shape
    T = target_boxes.shape[1]
    G = _G

    pad = jnp.broadcast_to(
        jnp.array([0.0, 0.0, -1.0, -1.0], dtype=pred_boxes.dtype),
        (B, _QP - Q, 4))
    pb_pad = jnp.concatenate([pred_boxes, pad], axis=1)
    pgrid = pb_pad.transpose(0, 2, 1).reshape(B, 4, _QR, _QL)
    l0grid = jnp.pad(pred_logits[:, :, 0], ((0, 0), (0, _QP - Q))
                     ).reshape(B, _QR, _QL)
    tflat = target_boxes.reshape(B // G, G, T * 4)
    labels = target_labels.astype(jnp.int32).reshape(B // G, G, T)

    ix = lambda i: (i, 0, 0)
    ix4 = lambda i: (i, 0, 0, 0)
    out = pl.pallas_call(
        _loss_kernel,
        out_shape=jax.ShapeDtypeStruct((B // G, 1, 2), jnp.float32),
        grid=(B // G,),
        in_specs=[
            pl.BlockSpec((G, Q, C), ix),
            pl.BlockSpec((G, 4, _QR, _QL), ix4),
            pl.BlockSpec((G, Q, 4), ix),
            pl.BlockSpec((G, T, 4), ix),
            pl.BlockSpec((G, _QR, _QL), ix),
            pl.BlockSpec((1, G, T * 4), ix, memory_space=pltpu.SMEM),
            pl.BlockSpec((1, G, T), ix, memory_space=pltpu.SMEM),
        ],
        out_specs=pl.BlockSpec((1, 1, 2), ix),
        scratch_shapes=([pltpu.VMEM((1, C), jnp.float32) for _ in range(G)]
                        + [pltpu.VMEM((1, 4), jnp.float32)
                           for _ in range(G)]),
        compiler_params=pltpu.CompilerParams(
            dimension_semantics=("parallel",),
            vmem_limit_bytes=56 * 1024 * 1024,
        ),
        name="detection_loss",
    )(pred_logits, pgrid, pred_boxes, target_boxes, l0grid, tflat, labels)

    cls_sum = jnp.sum(out[:, 0, 0])
    reg_sum = jnp.sum(out[:, 0, 1])
    return (_ALPHA * cls_sum + _BETA * reg_sum) / B


# speedup vs baseline: 1.0737x; 1.0737x over previous
import jax
import jax.numpy as jnp
from jax import lax
from jax.experimental import pallas as pl
from jax.experimental.pallas import tpu as pltpu

_IOU_THRESH = 0.1
_ALPHA = 1.0
_BETA = 1.0
_G = 8
_QR = 16
_QL = 128
_QP = _QR * _QL


def _loss_kernel(logits_ref, pgrid_ref, pbox_ref, tbox_ref, l0_ref,
                 tflat_ref, lbl_ref, out_ref, *accs):
    G, Q, C = logits_ref.shape
    T = tbox_ref.shape[1]
    ccls_s = accs[:G]
    creg_s = accs[G:]
    tflat = tflat_ref.at[0]
    lbls = lbl_ref.at[0]

    idx_grid = (lax.broadcasted_iota(jnp.int32, (_QR, _QL), 0) * _QL
                + lax.broadcasted_iota(jnp.int32, (_QR, _QL), 1))
    lane_c = lax.broadcasted_iota(jnp.int32, (1, C), 1)

    ce0 = []
    chunk = 360 if Q % 360 == 0 else Q
    for g in range(G):
        tot = jnp.zeros((1, 1), dtype=jnp.float32)
        for q0 in range(0, Q, chunk):
            x = logits_ref[g, q0:q0 + chunk, :]
            m = jnp.max(x, axis=1, keepdims=True)
            s = jnp.sum(jnp.exp(x - m), axis=1, keepdims=True)
            lse = m + jnp.log(s)
            tot = tot + jnp.sum(lse - x[:, 0:1], keepdims=True)
        ce0.append(tot)

    for g in range(G):
        ccls_s[g][...] = jnp.zeros_like(ccls_s[g])
        creg_s[g][...] = jnp.zeros_like(creg_s[g])

    def _iou_at(t):
        tb = 4 * t
        iou_l = []
        for g in range(G):
            tx1 = tflat[g, tb]
            ty1 = tflat[g, tb + 1]
            tx2 = tflat[g, tb + 2]
            ty2 = tflat[g, tb + 3]
            area_t = (tx2 - tx1) * (ty2 - ty1)

            px1 = pgrid_ref[g, 0]
            py1 = pgrid_ref[g, 1]
            px2 = pgrid_ref[g, 2]
            py2 = pgrid_ref[g, 3]
            iw = jnp.maximum(jnp.minimum(px2, tx2)
                             - jnp.maximum(px1, tx1), 0.0)
            ih = jnp.maximum(jnp.minimum(py2, ty2)
                             - jnp.maximum(py1, ty1), 0.0)
            inter = iw * ih
            union = (px2 - px1) * (py2 - py1) + area_t - inter
            iou_l.append(inter / (union + 1e-6))
        return jnp.stack(iou_l, axis=0)

    def _tree16(x3, op):
        a = op(x3[:, 0:8], x3[:, 8:16])
        b = op(a[:, 0:4], a[:, 4:8])
        c = op(b[:, 0:2], b[:, 2:4])
        return op(c[:, 0], c[:, 1])

    def body(t, carry):
        avail3, iou3 = carry
        masked3 = jnp.where(avail3 > 0.5, iou3, 0.0)
        iou_next = _iou_at(jnp.minimum(t + 1, T - 1))
        bmax = jnp.max(_tree16(masked3, jnp.maximum), axis=1,
                       keepdims=True)
        matched = bmax > _IOU_THRESH
        cand = masked3 == bmax[:, :, None]
        idxm = jnp.where(cand, idx_grid[None], _QP)
        qmin = jnp.min(_tree16(idxm, jnp.minimum), axis=1,
                       keepdims=True)
        qeff = jnp.where(matched, qmin, _QP)
        avail_new = jnp.where(idx_grid[None] == qeff[:, :, None],
                              0.0, avail3)

        for g in range(G):
            q = jnp.minimum(qmin[g, 0], Q - 1)
            mg = matched[g:g + 1, 0:1]
            lbl = lbls[g, t]
            row = logits_ref[g, pl.ds(q, 1), :]
            contrib = jnp.where(lane_c == lbl, row, 0.0)
            ccls_s[g][...] = ccls_s[g][...] - jnp.where(mg, contrib, 0.0)

            bp = pbox_ref[g, pl.ds(q, 1), :]
            tbx = tbox_ref[g, pl.ds(t, 1), :]
            d = jnp.abs(bp - tbx)
            sl1 = jnp.where(d < 1.0, 0.5 * d * d, d - 0.5)
            creg_s[g][...] = creg_s[g][...] + jnp.where(mg, sl1, 0.0)
        return (avail_new, iou_next)

    init = (jnp.ones((G, _QR, _QL), dtype=jnp.float32), _iou_at(0))
    avail3_fin, _ = lax.fori_loop(0, T, body, init, unroll=2)
    avail_fin = [avail3_fin[g] for g in range(G)]

    cls_tot = jnp.zeros((1, 1), dtype=jnp.float32)
    reg_tot = jnp.zeros((1, 1), dtype=jnp.float32)
    for g in range(G):
        l0_assigned = jnp.sum(
            jnp.where(avail_fin[g] < 0.5, l0_ref[g], 0.0), keepdims=True)
        cls_tot = (cls_tot + ce0[g] + l0_assigned
                   + jnp.sum(ccls_s[g][...], axis=1, keepdims=True))
        reg_tot = reg_tot + jnp.sum(creg_s[g][...], axis=1,
                                    keepdims=True) * 0.25
    out_ref[0] = jnp.concatenate([cls_tot, reg_tot], axis=1)


@jax.jit
def kernel(pred_logits, pred_boxes, target_boxes, target_labels):
    B, Q, C = pred_logits.shape
    T = target_boxes.shape[1]
    G = _G

    pad = jnp.broadcast_to(
        jnp.array([0.0, 0.0, -1.0, -1.0], dtype=pred_boxes.dtype),
        (B, _QP - Q, 4))
    pb_pad = jnp.concatenate([pred_boxes, pad], axis=1)
    pgrid = pb_pad.transpose(0, 2, 1).reshape(B, 4, _QR, _QL)
    l0grid = jnp.pad(pred_logits[:, :, 0], ((0, 0), (0, _QP - Q))
                     ).reshape(B, _QR, _QL)
    tflat = target_boxes.reshape(B // G, G, T * 4)
    labels = target_labels.astype(jnp.int32).reshape(B // G, G, T)

    ix = lambda i: (i, 0, 0)
    ix4 = lambda i: (i, 0, 0, 0)
    out = pl.pallas_call(
        _loss_kernel,
        out_shape=jax.ShapeDtypeStruct((B // G, 1, 2), jnp.float32),
        grid=(B // G,),
        in_specs=[
            pl.BlockSpec((G, Q, C), ix),
            pl.BlockSpec((G, 4, _QR, _QL), ix4),
            pl.BlockSpec((G, Q, 4), ix),
            pl.BlockSpec((G, T, 4), ix),
            pl.BlockSpec((G, _QR, _QL), ix),
            pl.BlockSpec((1, G, T * 4), ix, memory_space=pltpu.SMEM),
            pl.BlockSpec((1, G, T), ix, memory_space=pltpu.SMEM),
        ],
        out_specs=pl.BlockSpec((1, 1, 2), ix),
        scratch_shapes=([pltpu.VMEM((1, C), jnp.float32) for _ in range(G)]
                        + [pltpu.VMEM((1, 4), jnp.float32)
                           for _ in range(G)]),
        compiler_params=pltpu.CompilerParams(
            dimension_semantics=("parallel",),
            vmem_limit_bytes=56 * 1024 * 1024,
        ),
        name="detection_loss",
    )(pred_logits, pgrid, pred_boxes, target_boxes, l0grid, tflat, labels)

    cls_sum = jnp.sum(out[:, 0, 0])
    reg_sum = jnp.sum(out[:, 0, 1])
    return (_ALPHA * cls_sum + _BETA * reg_sum) / B


# speedup vs baseline: 1.1221x; 1.0451x over previous
import jax
import jax.numpy as jnp
from jax import lax
from jax.experimental import pallas as pl
from jax.experimental.pallas import tpu as pltpu

_IOU_THRESH = 0.1
_ALPHA = 1.0
_BETA = 1.0
_G = 8
_QR = 16
_QL = 128
_QP = _QR * _QL


def _loss_kernel(logits_ref, pgrid_ref, pbox_ref, tbox_ref, l0_ref,
                 tflat_ref, lbl_ref, out_ref, *accs):
    G, Q, C = logits_ref.shape
    T = tbox_ref.shape[1]
    ccls_s = accs[:G]
    creg_s = accs[G:]
    tflat = tflat_ref.at[0]
    lbls = lbl_ref.at[0]

    idx_grid = (lax.broadcasted_iota(jnp.int32, (_QR, _QL), 0) * _QL
                + lax.broadcasted_iota(jnp.int32, (_QR, _QL), 1))
    lane_c = lax.broadcasted_iota(jnp.int32, (1, C), 1)

    ce0 = []
    chunk = 360 if Q % 360 == 0 else Q
    for g in range(G):
        tot = jnp.zeros((1, 1), dtype=jnp.float32)
        for q0 in range(0, Q, chunk):
            x = logits_ref[g, q0:q0 + chunk, :]
            m = jnp.max(x, axis=1, keepdims=True)
            s = jnp.sum(jnp.exp(x - m), axis=1, keepdims=True)
            lse = m + jnp.log(s)
            tot = tot + jnp.sum(lse - x[:, 0:1], keepdims=True)
        ce0.append(tot)

    for g in range(G):
        ccls_s[g][...] = jnp.zeros_like(ccls_s[g])
        creg_s[g][...] = jnp.zeros_like(creg_s[g])

    def _iou_at(t):
        tb = 4 * t
        iou_l = []
        for g in range(G):
            tx1 = tflat[g, tb]
            ty1 = tflat[g, tb + 1]
            tx2 = tflat[g, tb + 2]
            ty2 = tflat[g, tb + 3]
            area_t = (tx2 - tx1) * (ty2 - ty1)

            px1 = pgrid_ref[g, 0]
            py1 = pgrid_ref[g, 1]
            px2 = pgrid_ref[g, 2]
            py2 = pgrid_ref[g, 3]
            iw = jnp.maximum(jnp.minimum(px2, tx2)
                             - jnp.maximum(px1, tx1), 0.0)
            ih = jnp.maximum(jnp.minimum(py2, ty2)
                             - jnp.maximum(py1, ty1), 0.0)
            inter = iw * ih
            union = (px2 - px1) * (py2 - py1) + area_t - inter
            iou_l.append(inter / (union + 1e-6))
        return jnp.stack(iou_l, axis=0)

    def _tree16(x3, op):
        a = op(x3[:, 0:8], x3[:, 8:16])
        b = op(a[:, 0:4], a[:, 4:8])
        c = op(b[:, 0:2], b[:, 2:4])
        return op(c[:, 0], c[:, 1])

    def body(t, carry):
        avail3, iou3 = carry
        masked3 = jnp.where(avail3 > 0.5, iou3, 0.0)
        iou_next = _iou_at(jnp.minimum(t + 1, T - 1))
        bmax = jnp.max(_tree16(masked3, jnp.maximum), axis=1,
                       keepdims=True)
        matched = bmax > _IOU_THRESH
        cand = masked3 == bmax[:, :, None]
        idxm = jnp.where(cand, idx_grid[None], _QP)
        qmin = jnp.min(_tree16(idxm, jnp.minimum), axis=1,
                       keepdims=True)
        qeff = jnp.where(matched, qmin, _QP)
        avail_new = jnp.where(idx_grid[None] == qeff[:, :, None],
                              0.0, avail3)

        for g in range(G):
            q = jnp.minimum(qmin[g, 0], Q - 1)
            mg = matched[g:g + 1, 0:1]
            lbl = lbls[g, t]
            row = logits_ref[g, pl.ds(q, 1), :]
            contrib = jnp.where(lane_c == lbl, row, 0.0)
            ccls_s[g][...] = ccls_s[g][...] - jnp.where(mg, contrib, 0.0)

            bp = pbox_ref[g, pl.ds(q, 1), :]
            tbx = tbox_ref[g, pl.ds(t, 1), :]
            d = jnp.abs(bp - tbx)
            sl1 = jnp.where(d < 1.0, 0.5 * d * d, d - 0.5)
            creg_s[g][...] = creg_s[g][...] + jnp.where(mg, sl1, 0.0)
        return (avail_new, iou_next)

    init = (jnp.ones((G, _QR, _QL), dtype=jnp.float32), _iou_at(0))
    avail3_fin, _ = lax.fori_loop(0, T, body, init, unroll=4)
    avail_fin = [avail3_fin[g] for g in range(G)]

    cls_tot = jnp.zeros((1, 1), dtype=jnp.float32)
    reg_tot = jnp.zeros((1, 1), dtype=jnp.float32)
    for g in range(G):
        l0_assigned = jnp.sum(
            jnp.where(avail_fin[g] < 0.5, l0_ref[g], 0.0), keepdims=True)
        cls_tot = (cls_tot + ce0[g] + l0_assigned
                   + jnp.sum(ccls_s[g][...], axis=1, keepdims=True))
        reg_tot = reg_tot + jnp.sum(creg_s[g][...], axis=1,
                                    keepdims=True) * 0.25
    out_ref[0] = jnp.concatenate([cls_tot, reg_tot], axis=1)


@jax.jit
def kernel(pred_logits, pred_boxes, target_boxes, target_labels):
    B, Q, C = pred_logits.shape
    T = target_boxes.shape[1]
    G = _G

    pad = jnp.broadcast_to(
        jnp.array([0.0, 0.0, -1.0, -1.0], dtype=pred_boxes.dtype),
        (B, _QP - Q, 4))
    pb_pad = jnp.concatenate([pred_boxes, pad], axis=1)
    pgrid = pb_pad.transpose(0, 2, 1).reshape(B, 4, _QR, _QL)
    l0grid = jnp.pad(pred_logits[:, :, 0], ((0, 0), (0, _QP - Q))
                     ).reshape(B, _QR, _QL)
    tflat = target_boxes.reshape(B // G, G, T * 4)
    labels = target_labels.astype(jnp.int32).reshape(B // G, G, T)

    ix = lambda i: (i, 0, 0)
    ix4 = lambda i: (i, 0, 0, 0)
    out = pl.pallas_call(
        _loss_kernel,
        out_shape=jax.ShapeDtypeStruct((B // G, 1, 2), jnp.float32),
        grid=(B // G,),
        in_specs=[
            pl.BlockSpec((G, Q, C), ix),
            pl.BlockSpec((G, 4, _QR, _QL), ix4),
            pl.BlockSpec((G, Q, 4), ix),
            pl.BlockSpec((G, T, 4), ix),
            pl.BlockSpec((G, _QR, _QL), ix),
            pl.BlockSpec((1, G, T * 4), ix, memory_space=pltpu.SMEM),
            pl.BlockSpec((1, G, T), ix, memory_space=pltpu.SMEM),
        ],
        out_specs=pl.BlockSpec((1, 1, 2), ix),
        scratch_shapes=([pltpu.VMEM((1, C), jnp.float32) for _ in range(G)]
                        + [pltpu.VMEM((1, 4), jnp.float32)
                           for _ in range(G)]),
        compiler_params=pltpu.CompilerParams(
            dimension_semantics=("parallel",),
            vmem_limit_bytes=56 * 1024 * 1024,
        ),
        name="detection_loss",
    )(pred_logits, pgrid, pred_boxes, target_boxes, l0grid, tflat, labels)

    cls_sum = jnp.sum(out[:, 0, 0])
    reg_sum = jnp.sum(out[:, 0, 1])
    return (_ALPHA * cls_sum + _BETA * reg_sum) / B
